# Initial kernel scaffold; baseline (speedup 1.0000x reference)
import jax
import jax.numpy as jnp
from jax.experimental import pallas as pl


def kernel(x, edge_index, W_gcn, b_gcn, grid1, base_w1, spline_w1, scaler1, grid2, base_w2, spline_w2, scaler2):
    raise NotImplementedError("write your pallas kernel here")


# speedup vs baseline: 22.8520x; 22.8520x over previous
import functools

import jax
import jax.numpy as jnp
from jax import lax
from jax.experimental import pallas as pl
from jax.experimental.pallas import tpu as pltpu
from jax.experimental.pallas import tpu_sc as plsc

_NC = 2
_NS = 16
_NW = _NC * _NS
_L = 16

_SPLINE_ORDER = 3
_EDGE_CHUNK = 80


def _make_deg_kernel(n_nodes, n_edges):
    rows_pw = n_edges // _NW // _L
    nrow = n_nodes // _L
    mesh = plsc.VectorSubcoreMesh(core_axis_name="c", subcore_axis_name="s")

    @functools.partial(
        pl.kernel,
        mesh=mesh,
        out_type=jax.ShapeDtypeStruct((_NW, nrow, _L), jnp.float32),
        compiler_params=pltpu.CompilerParams(needs_layout_passes=False, use_tc_tiling_on_sc=False),
        scratch_types=[
            pltpu.VMEM((rows_pw, _L), jnp.int32),
            pltpu.VMEM((nrow, _L), jnp.float32),
        ],
    )
    def deg_kernel(dst_hbm, out_hbm, idx_v, hist_v):
        c = lax.axis_index("c")
        s = lax.axis_index("s")
        w = c * _NS + s

        zeros16 = jnp.zeros((_L,), jnp.float32)

        def zero_body(i, carry):
            hist_v[i] = zeros16
            return carry

        lax.fori_loop(0, nrow, zero_body, 0)

        pltpu.sync_copy(dst_hbm.at[w], idx_v)

        ones16 = jnp.ones((_L,), jnp.float32)

        def body(i, carry):
            idx = idx_v[i]
            plsc.addupdate_scatter(
                hist_v,
                [lax.shift_right_logical(idx, 4), lax.bitwise_and(idx, 15)],
                ones16)
            return carry

        lax.fori_loop(0, rows_pw, body, 0)

        pltpu.sync_copy(hist_v, out_hbm.at[w])

    return deg_kernel


def _make_agg_kernel(n_nodes, n_edges, hid):
    ch = _EDGE_CHUNK
    nch = n_edges // _NW // ch
    nstage = 10
    rows_ps = n_nodes // nstage
    mesh = plsc.VectorSubcoreMesh(core_axis_name="c", subcore_axis_name="s")

    @functools.partial(
        pl.kernel,
        mesh=mesh,
        out_type=jax.ShapeDtypeStruct((_NC, n_nodes, hid), jnp.float32),
        compiler_params=pltpu.CompilerParams(needs_layout_passes=False, use_tc_tiling_on_sc=False),
        scratch_types=[
            pltpu.VMEM((nch, ch), jnp.int32),
            pltpu.VMEM((nch, ch), jnp.int32),
            pltpu.VMEM((ch, hid), jnp.float32),
            pltpu.VMEM_SHARED((n_nodes, hid), jnp.float32),
            pltpu.SemaphoreType.DMA,
        ],
    )
    def agg_kernel(src_hbm, dst_hbm, hs_hbm, out_hbm, src_v, dst_v, rows_v,
                   acc_sh, gsem):
        c = lax.axis_index("c")
        s = lax.axis_index("s")
        w = c * _NS + s

        pltpu.sync_copy(src_hbm.at[w], src_v)
        pltpu.sync_copy(dst_hbm.at[w], dst_v)

        @pl.when(s < nstage)
        def _():
            pltpu.sync_copy(hs_hbm.at[pl.ds(s * rows_ps, rows_ps)],
                            acc_sh.at[pl.ds(s * rows_ps, rows_ps)])
        plsc.subcore_barrier()

        def body(j, carry):
            pltpu.async_copy(hs_hbm.at[src_v.at[j]], rows_v, gsem).wait()
            pltpu.sync_copy(rows_v, acc_sh.at[dst_v.at[j]], add=True)
            return carry

        lax.fori_loop(0, nch, body, 0)

        plsc.subcore_barrier()

        @pl.when(s < nstage)
        def _():
            pltpu.sync_copy(acc_sh.at[pl.ds(s * rows_ps, rows_ps)],
                            out_hbm.at[c, pl.ds(s * rows_ps, rows_ps)])

    return agg_kernel


def _hs_body(x_ref, w_ref, degp_ref, hs_ref):
    deg = jnp.sum(degp_ref[...], axis=1) + 1.0
    dinv = lax.rsqrt(deg)
    h = lax.dot_general(x_ref[...], w_ref[...], (((1,), (1,)), ((), ())),
                        preferred_element_type=jnp.float32)
    hs_ref[...] = h * dinv[:, None]


def _kan_apply(x, grid_t, bw, sw_t, sc):
    base = lax.dot_general(jax.nn.silu(x), bw, (((1,), (1,)), ((), ())),
                           preferred_element_type=jnp.float32)
    ng = grid_t.shape[0]
    g = [grid_t[j][None, :] for j in range(ng)]
    bases = [jnp.where((x >= g[j]) & (x < g[j + 1]), 1.0, 0.0)
             for j in range(ng - 1)]
    for k in range(1, _SPLINE_ORDER + 1):
        nb = []
        for j in range(len(bases) - 1):
            left = (x - g[j]) / (g[j + k] - g[j]) * bases[j]
            right = (g[j + k + 1] - x) / (g[j + k + 1] - g[j + 1]) * bases[j + 1]
            nb.append(left + right)
        bases = nb
    out = base
    for j in range(len(bases)):
        sj = sw_t[j] * sc
        out = out + lax.dot_general(bases[j], sj, (((1,), (1,)), ((), ())),
                                    preferred_element_type=jnp.float32)
    return out


def _mlp_body(aggp_ref, hs_ref, degp_ref, b_ref, g1_ref, bw1_ref, sw1_ref,
              sc1_ref, g2_ref, bw2_ref, sw2_ref, sc2_ref, out_ref):
    deg = jnp.sum(degp_ref[...], axis=1) + 1.0
    dinv = lax.rsqrt(deg)
    agg = aggp_ref[0] + aggp_ref[1] - hs_ref[...]
    h1 = jnp.maximum(agg * dinv[:, None] + b_ref[...], 0.0)
    h2 = _kan_apply(h1, g1_ref[...], bw1_ref[...], sw1_ref[...], sc1_ref[...])
    h3 = _kan_apply(h2, g2_ref[...], bw2_ref[...], sw2_ref[...], sc2_ref[...])
    out_ref[...] = h3


def kernel(x, edge_index, W_gcn, b_gcn, grid1, base_w1, spline_w1, scaler1,
           grid2, base_w2, spline_w2, scaler2):
    n, in_ch = x.shape
    hid = W_gcn.shape[0]
    mid = base_w1.shape[0]
    out_ch = base_w2.shape[0]
    e = edge_index.shape[1]

    src = edge_index[0]
    dst = edge_index[1]

    degp = _make_deg_kernel(n, e)(
        dst.reshape(_NW, e // _NW // _L, _L)).reshape(_NW, n).T

    blk = 1000
    hs = pl.pallas_call(
        _hs_body,
        grid=(n // blk,),
        in_specs=[
            pl.BlockSpec((blk, in_ch), lambda i: (i, 0)),
            pl.BlockSpec((hid, in_ch), lambda i: (0, 0)),
            pl.BlockSpec((blk, _NW), lambda i: (i, 0)),
        ],
        out_specs=pl.BlockSpec((blk, hid), lambda i: (i, 0)),
        out_shape=jax.ShapeDtypeStruct((n, hid), jnp.float32),
    )(x, W_gcn, degp)

    ch = _EDGE_CHUNK
    nch = e // _NW // ch
    aggp = _make_agg_kernel(n, e, hid)(
        src.reshape(_NW, nch, ch), dst.reshape(_NW, nch, ch), hs)

    g1_t = grid1.T
    g2_t = grid2.T
    sw1_t = jnp.transpose(spline_w1, (2, 0, 1))
    sw2_t = jnp.transpose(spline_w2, (2, 0, 1))
    b2d = b_gcn[None, :]
    nk1 = sw1_t.shape[0]
    nk2 = sw2_t.shape[0]
    ng1 = g1_t.shape[0]
    ng2 = g2_t.shape[0]

    out = pl.pallas_call(
        _mlp_body,
        grid=(n // blk,),
        in_specs=[
            pl.BlockSpec((_NC, blk, hid), lambda i: (0, i, 0)),
            pl.BlockSpec((blk, hid), lambda i: (i, 0)),
            pl.BlockSpec((blk, _NW), lambda i: (i, 0)),
            pl.BlockSpec((1, hid), lambda i: (0, 0)),
            pl.BlockSpec((ng1, hid), lambda i: (0, 0)),
            pl.BlockSpec((mid, hid), lambda i: (0, 0)),
            pl.BlockSpec((nk1, mid, hid), lambda i: (0, 0, 0)),
            pl.BlockSpec((mid, hid), lambda i: (0, 0)),
            pl.BlockSpec((ng2, mid), lambda i: (0, 0)),
            pl.BlockSpec((out_ch, mid), lambda i: (0, 0)),
            pl.BlockSpec((nk2, out_ch, mid), lambda i: (0, 0, 0)),
            pl.BlockSpec((out_ch, mid), lambda i: (0, 0)),
        ],
        out_specs=pl.BlockSpec((blk, out_ch), lambda i: (i, 0)),
        out_shape=jax.ShapeDtypeStruct((n, out_ch), jnp.float32),
    )(aggp, hs, degp, b2d, g1_t, base_w1, sw1_t, scaler1,
      g2_t, base_w2, sw2_t, scaler2)

    return out


# speedup vs baseline: 31.4619x; 1.3768x over previous
import functools

import jax
import jax.numpy as jnp
from jax import lax
from jax.experimental import pallas as pl
from jax.experimental.pallas import tpu as pltpu
from jax.experimental.pallas import tpu_sc as plsc

_NC = 2
_NS = 16
_NW = _NC * _NS
_L = 16

_SPLINE_ORDER = 3
_EDGE_CHUNK = 80


def _make_deg_kernel(n_nodes, n_edges):
    rows_pw = n_edges // _NW // _L
    nrow = n_nodes // _L
    mesh = plsc.VectorSubcoreMesh(core_axis_name="c", subcore_axis_name="s")

    @functools.partial(
        pl.kernel,
        mesh=mesh,
        out_type=jax.ShapeDtypeStruct((_NW, nrow, _L), jnp.float32),
        compiler_params=pltpu.CompilerParams(needs_layout_passes=False, use_tc_tiling_on_sc=False),
        scratch_types=[
            pltpu.VMEM((rows_pw, _L), jnp.int32),
            pltpu.VMEM((nrow, _L), jnp.float32),
        ],
    )
    def deg_kernel(dst_hbm, out_hbm, idx_v, hist_v):
        c = lax.axis_index("c")
        s = lax.axis_index("s")
        w = c * _NS + s

        zeros16 = jnp.zeros((_L,), jnp.float32)

        def zero_body(i, carry):
            hist_v[i] = zeros16
            return carry

        lax.fori_loop(0, nrow, zero_body, 0)

        pltpu.sync_copy(dst_hbm.at[w], idx_v)

        ones16 = jnp.ones((_L,), jnp.float32)

        def body(i, carry):
            idx = idx_v[i]
            plsc.addupdate_scatter(
                hist_v,
                [lax.shift_right_logical(idx, 4), lax.bitwise_and(idx, 15)],
                ones16)
            return carry

        lax.fori_loop(0, rows_pw, body, 0)

        pltpu.sync_copy(hist_v, out_hbm.at[w])

    return deg_kernel


_NBUF = 5


def _make_agg_kernel(n_nodes, n_edges, hid):
    ch = _EDGE_CHUNK
    nch = n_edges // _NW // ch
    assert nch % _NBUF == 0
    ngrp = nch // _NBUF
    nstage = 10
    rows_ps = n_nodes // nstage
    mesh = plsc.VectorSubcoreMesh(core_axis_name="c", subcore_axis_name="s")

    @functools.partial(
        pl.kernel,
        mesh=mesh,
        out_type=jax.ShapeDtypeStruct((_NC, n_nodes, hid), jnp.float32),
        compiler_params=pltpu.CompilerParams(needs_layout_passes=False, use_tc_tiling_on_sc=False),
        scratch_types=[
            pltpu.VMEM((nch, ch), jnp.int32),
            pltpu.VMEM((nch, ch), jnp.int32),
            pltpu.VMEM((_NBUF, ch, hid), jnp.float32),
            pltpu.VMEM_SHARED((n_nodes, hid), jnp.float32),
        ] + [pltpu.SemaphoreType.DMA] * (2 * _NBUF),
    )
    def agg_kernel(src_hbm, dst_hbm, hs_hbm, out_hbm, src_v, dst_v, rows_v,
                   acc_sh, *sems):
        gsem = sems[:_NBUF]
        ssem = sems[_NBUF:]
        c = lax.axis_index("c")
        s = lax.axis_index("s")
        w = c * _NS + s

        pltpu.sync_copy(src_hbm.at[w], src_v)
        pltpu.sync_copy(dst_hbm.at[w], dst_v)

        @pl.when(s < nstage)
        def _():
            pltpu.sync_copy(hs_hbm.at[pl.ds(s * rows_ps, rows_ps)],
                            acc_sh.at[pl.ds(s * rows_ps, rows_ps)])
        plsc.subcore_barrier()

        def wait_gather(b):
            pltpu.make_async_copy(hs_hbm.at[pl.ds(0, ch)], rows_v.at[b],
                                  gsem[b]).wait()

        def wait_scatter(b):
            pltpu.make_async_copy(rows_v.at[b], acc_sh.at[pl.ds(0, ch)],
                                  ssem[b]).wait()

        for b in range(_NBUF):
            pltpu.async_copy(hs_hbm.at[src_v.at[b]], rows_v.at[b], gsem[b])

        def body(t, carry):
            base = t * _NBUF
            for b in range(_NBUF):
                wait_gather(b)
                pltpu.async_copy(rows_v.at[b], acc_sh.at[dst_v.at[base + b]],
                                 ssem[b], add=True)
            for b in range(_NBUF):
                jn = base + _NBUF + b

                @pl.when(jn < nch)
                def _(b=b, jn=jn):
                    wait_scatter(b)
                    pltpu.async_copy(hs_hbm.at[src_v.at[jn]], rows_v.at[b],
                                     gsem[b])
            return carry

        lax.fori_loop(0, ngrp, body, 0)

        for b in range(_NBUF):
            wait_scatter(b)

        plsc.subcore_barrier()

        @pl.when(s < nstage)
        def _():
            pltpu.sync_copy(acc_sh.at[pl.ds(s * rows_ps, rows_ps)],
                            out_hbm.at[c, pl.ds(s * rows_ps, rows_ps)])

    return agg_kernel


def _hs_body(x_ref, w_ref, degp_ref, hs_ref):
    deg = jnp.sum(degp_ref[...], axis=1) + 1.0
    dinv = lax.rsqrt(deg)
    h = lax.dot_general(x_ref[...], w_ref[...], (((1,), (1,)), ((), ())),
                        preferred_element_type=jnp.float32)
    hs_ref[...] = h * dinv[:, None]


def _kan_apply(x, grid_t, bw, sw_t, sc):
    base = lax.dot_general(jax.nn.silu(x), bw, (((1,), (1,)), ((), ())),
                           preferred_element_type=jnp.float32)
    ng = grid_t.shape[0]
    g = [grid_t[j][None, :] for j in range(ng)]
    bases = [jnp.where((x >= g[j]) & (x < g[j + 1]), 1.0, 0.0)
             for j in range(ng - 1)]
    for k in range(1, _SPLINE_ORDER + 1):
        nb = []
        for j in range(len(bases) - 1):
            left = (x - g[j]) / (g[j + k] - g[j]) * bases[j]
            right = (g[j + k + 1] - x) / (g[j + k + 1] - g[j + 1]) * bases[j + 1]
            nb.append(left + right)
        bases = nb
    out = base
    for j in range(len(bases)):
        sj = sw_t[j] * sc
        out = out + lax.dot_general(bases[j], sj, (((1,), (1,)), ((), ())),
                                    preferred_element_type=jnp.float32)
    return out


def _mlp_body(aggp_ref, hs_ref, degp_ref, b_ref, g1_ref, bw1_ref, sw1_ref,
              sc1_ref, g2_ref, bw2_ref, sw2_ref, sc2_ref, out_ref):
    deg = jnp.sum(degp_ref[...], axis=1) + 1.0
    dinv = lax.rsqrt(deg)
    agg = aggp_ref[0] + aggp_ref[1] - hs_ref[...]
    h1 = jnp.maximum(agg * dinv[:, None] + b_ref[...], 0.0)
    h2 = _kan_apply(h1, g1_ref[...], bw1_ref[...], sw1_ref[...], sc1_ref[...])
    h3 = _kan_apply(h2, g2_ref[...], bw2_ref[...], sw2_ref[...], sc2_ref[...])
    out_ref[...] = h3


def kernel(x, edge_index, W_gcn, b_gcn, grid1, base_w1, spline_w1, scaler1,
           grid2, base_w2, spline_w2, scaler2):
    n, in_ch = x.shape
    hid = W_gcn.shape[0]
    mid = base_w1.shape[0]
    out_ch = base_w2.shape[0]
    e = edge_index.shape[1]

    src = edge_index[0]
    dst = edge_index[1]

    degp = _make_deg_kernel(n, e)(
        dst.reshape(_NW, e // _NW // _L, _L)).reshape(_NW, n).T

    blk = 1000
    hs = pl.pallas_call(
        _hs_body,
        grid=(n // blk,),
        in_specs=[
            pl.BlockSpec((blk, in_ch), lambda i: (i, 0)),
            pl.BlockSpec((hid, in_ch), lambda i: (0, 0)),
            pl.BlockSpec((blk, _NW), lambda i: (i, 0)),
        ],
        out_specs=pl.BlockSpec((blk, hid), lambda i: (i, 0)),
        out_shape=jax.ShapeDtypeStruct((n, hid), jnp.float32),
    )(x, W_gcn, degp)

    ch = _EDGE_CHUNK
    nch = e // _NW // ch
    aggp = _make_agg_kernel(n, e, hid)(
        src.reshape(_NW, nch, ch), dst.reshape(_NW, nch, ch), hs)

    g1_t = grid1.T
    g2_t = grid2.T
    sw1_t = jnp.transpose(spline_w1, (2, 0, 1))
    sw2_t = jnp.transpose(spline_w2, (2, 0, 1))
    b2d = b_gcn[None, :]
    nk1 = sw1_t.shape[0]
    nk2 = sw2_t.shape[0]
    ng1 = g1_t.shape[0]
    ng2 = g2_t.shape[0]

    out = pl.pallas_call(
        _mlp_body,
        grid=(n // blk,),
        in_specs=[
            pl.BlockSpec((_NC, blk, hid), lambda i: (0, i, 0)),
            pl.BlockSpec((blk, hid), lambda i: (i, 0)),
            pl.BlockSpec((blk, _NW), lambda i: (i, 0)),
            pl.BlockSpec((1, hid), lambda i: (0, 0)),
            pl.BlockSpec((ng1, hid), lambda i: (0, 0)),
            pl.BlockSpec((mid, hid), lambda i: (0, 0)),
            pl.BlockSpec((nk1, mid, hid), lambda i: (0, 0, 0)),
            pl.BlockSpec((mid, hid), lambda i: (0, 0)),
            pl.BlockSpec((ng2, mid), lambda i: (0, 0)),
            pl.BlockSpec((out_ch, mid), lambda i: (0, 0)),
            pl.BlockSpec((nk2, out_ch, mid), lambda i: (0, 0, 0)),
            pl.BlockSpec((out_ch, mid), lambda i: (0, 0)),
        ],
        out_specs=pl.BlockSpec((blk, out_ch), lambda i: (i, 0)),
        out_shape=jax.ShapeDtypeStruct((n, out_ch), jnp.float32),
    )(aggp, hs, degp, b2d, g1_t, base_w1, sw1_t, scaler1,
      g2_t, base_w2, sw2_t, scaler2)

    return out


# speedup vs baseline: 34.6037x; 1.0999x over previous
import functools

import jax
import jax.numpy as jnp
from jax import lax
from jax.experimental import pallas as pl
from jax.experimental.pallas import tpu as pltpu
from jax.experimental.pallas import tpu_sc as plsc

_NC = 2
_NS = 16
_NW = _NC * _NS
_L = 16

_SPLINE_ORDER = 3
_EDGE_CHUNK = 80


def _make_deg_kernel(n_nodes, n_edges):
    rows_pw = n_edges // _NW // _L
    nrow = n_nodes // _L
    mesh = plsc.VectorSubcoreMesh(core_axis_name="c", subcore_axis_name="s")

    @functools.partial(
        pl.kernel,
        mesh=mesh,
        out_type=jax.ShapeDtypeStruct((_NW, nrow, _L), jnp.float32),
        compiler_params=pltpu.CompilerParams(needs_layout_passes=False, use_tc_tiling_on_sc=False),
        scratch_types=[
            pltpu.VMEM((rows_pw, _L), jnp.int32),
            pltpu.VMEM((nrow, _L), jnp.float32),
        ],
    )
    def deg_kernel(dst_hbm, out_hbm, idx_v, hist_v):
        c = lax.axis_index("c")
        s = lax.axis_index("s")
        w = c * _NS + s

        zeros16 = jnp.zeros((_L,), jnp.float32)

        def zero_body(i, carry):
            hist_v[i] = zeros16
            return carry

        lax.fori_loop(0, nrow, zero_body, 0)

        pltpu.sync_copy(dst_hbm.at[w], idx_v)

        ones16 = jnp.ones((_L,), jnp.float32)

        def body(i, carry):
            idx = idx_v[i]
            plsc.addupdate_scatter(
                hist_v,
                [lax.shift_right_logical(idx, 4), lax.bitwise_and(idx, 15)],
                ones16)
            return carry

        lax.fori_loop(0, rows_pw, body, 0)

        pltpu.sync_copy(hist_v, out_hbm.at[w])

    return deg_kernel


_NBUF = 5


def _make_agg_kernel(n_nodes, n_edges, hid):
    ch = _EDGE_CHUNK
    nch = n_edges // _NW // ch
    assert nch % _NBUF == 0
    ngrp = nch // _NBUF
    nstage = 10
    rows_ps = n_nodes // nstage
    mesh = plsc.VectorSubcoreMesh(core_axis_name="c", subcore_axis_name="s")

    @functools.partial(
        pl.kernel,
        mesh=mesh,
        out_type=jax.ShapeDtypeStruct((_NC, n_nodes, hid), jnp.float32),
        compiler_params=pltpu.CompilerParams(needs_layout_passes=False, use_tc_tiling_on_sc=False),
        scratch_types=[
            pltpu.VMEM((nch, ch), jnp.int32),
            pltpu.VMEM((nch, ch), jnp.int32),
            pltpu.VMEM((_NBUF, ch, hid), jnp.float32),
            pltpu.VMEM_SHARED((n_nodes, hid), jnp.float32),
        ] + [pltpu.SemaphoreType.DMA] * (2 * _NBUF),
    )
    def agg_kernel(src_hbm, dst_hbm, hs_hbm, out_hbm, src_v, dst_v, rows_v,
                   acc_sh, *sems):
        gsem = sems[:_NBUF]
        ssem = sems[_NBUF:]
        c = lax.axis_index("c")
        s = lax.axis_index("s")
        w = c * _NS + s

        pltpu.sync_copy(src_hbm.at[w], src_v)
        pltpu.sync_copy(dst_hbm.at[w], dst_v)

        @pl.when(s < nstage)
        def _():
            pltpu.sync_copy(hs_hbm.at[pl.ds(s * rows_ps, rows_ps)],
                            acc_sh.at[pl.ds(s * rows_ps, rows_ps)])
        plsc.subcore_barrier()

        def wait_gather(b):
            pltpu.make_async_copy(hs_hbm.at[pl.ds(0, ch)], rows_v.at[b],
                                  gsem[b]).wait()

        def wait_scatter(b):
            pltpu.make_async_copy(rows_v.at[b], acc_sh.at[pl.ds(0, ch)],
                                  ssem[b]).wait()

        for b in range(_NBUF):
            pltpu.async_copy(hs_hbm.at[src_v.at[b]], rows_v.at[b], gsem[b])

        def body(t, carry):
            base = t * _NBUF
            for b in range(_NBUF):
                wait_gather(b)
                pltpu.async_copy(rows_v.at[b], acc_sh.at[dst_v.at[base + b]],
                                 ssem[b], add=True)
            for b in range(_NBUF):
                jn = base + _NBUF + b

                @pl.when(jn < nch)
                def _(b=b, jn=jn):
                    wait_scatter(b)
                    pltpu.async_copy(hs_hbm.at[src_v.at[jn]], rows_v.at[b],
                                     gsem[b])
            return carry

        lax.fori_loop(0, ngrp, body, 0)

        for b in range(_NBUF):
            wait_scatter(b)

        plsc.subcore_barrier()

        @pl.when(s < nstage)
        def _():
            pltpu.sync_copy(acc_sh.at[pl.ds(s * rows_ps, rows_ps)],
                            out_hbm.at[c, pl.ds(s * rows_ps, rows_ps)])

    return agg_kernel


def _hs_body(x_ref, w_ref, degp_ref, hs_ref):
    deg = jnp.sum(degp_ref[...], axis=1) + 1.0
    dinv = lax.rsqrt(deg)
    h = lax.dot_general(x_ref[...], w_ref[...], (((1,), (1,)), ((), ())),
                        preferred_element_type=jnp.float32)
    hs_ref[...] = h * dinv[:, None]


def _kan_apply(x, grid_t, bw, sw_t, sc):
    base = lax.dot_general(jax.nn.silu(x), bw, (((1,), (1,)), ((), ())),
                           preferred_element_type=jnp.float32)
    ng = grid_t.shape[0]
    g0 = grid_t[0][None, :]
    h = grid_t[1][None, :] - g0
    t = (x - g0) / h
    d = [t - float(j) for j in range(ng)]
    bases = [jnp.where((d[j] >= 0.0) & (d[j + 1] < 0.0), 1.0, 0.0)
             for j in range(ng - 1)]
    for k in range(1, _SPLINE_ORDER + 1):
        bases = [d[j] * bases[j] - d[j + k + 1] * bases[j + 1]
                 for j in range(len(bases) - 1)]
    scale = 1.0
    for k in range(1, _SPLINE_ORDER + 1):
        scale /= k
    out = base
    for j in range(len(bases)):
        sj = sw_t[j] * (sc * scale)
        out = out + lax.dot_general(bases[j], sj, (((1,), (1,)), ((), ())),
                                    preferred_element_type=jnp.float32)
    return out


def _mlp_body(aggp_ref, hs_ref, degp_ref, b_ref, g1_ref, bw1_ref, sw1_ref,
              sc1_ref, g2_ref, bw2_ref, sw2_ref, sc2_ref, out_ref):
    deg = jnp.sum(degp_ref[...], axis=1) + 1.0
    dinv = lax.rsqrt(deg)
    agg = aggp_ref[0] + aggp_ref[1] - hs_ref[...]
    h1 = jnp.maximum(agg * dinv[:, None] + b_ref[...], 0.0)
    h2 = _kan_apply(h1, g1_ref[...], bw1_ref[...], sw1_ref[...], sc1_ref[...])
    h3 = _kan_apply(h2, g2_ref[...], bw2_ref[...], sw2_ref[...], sc2_ref[...])
    out_ref[...] = h3


def kernel(x, edge_index, W_gcn, b_gcn, grid1, base_w1, spline_w1, scaler1,
           grid2, base_w2, spline_w2, scaler2):
    n, in_ch = x.shape
    hid = W_gcn.shape[0]
    mid = base_w1.shape[0]
    out_ch = base_w2.shape[0]
    e = edge_index.shape[1]

    src = edge_index[0]
    dst = edge_index[1]

    degp = _make_deg_kernel(n, e)(
        dst.reshape(_NW, e // _NW // _L, _L)).reshape(_NW, n).T

    blk = 1000
    hs = pl.pallas_call(
        _hs_body,
        grid=(n // blk,),
        in_specs=[
            pl.BlockSpec((blk, in_ch), lambda i: (i, 0)),
            pl.BlockSpec((hid, in_ch), lambda i: (0, 0)),
            pl.BlockSpec((blk, _NW), lambda i: (i, 0)),
        ],
        out_specs=pl.BlockSpec((blk, hid), lambda i: (i, 0)),
        out_shape=jax.ShapeDtypeStruct((n, hid), jnp.float32),
    )(x, W_gcn, degp)

    ch = _EDGE_CHUNK
    nch = e // _NW // ch
    aggp = _make_agg_kernel(n, e, hid)(
        src.reshape(_NW, nch, ch), dst.reshape(_NW, nch, ch), hs)

    g1_t = grid1.T
    g2_t = grid2.T
    sw1_t = jnp.transpose(spline_w1, (2, 0, 1))
    sw2_t = jnp.transpose(spline_w2, (2, 0, 1))
    b2d = b_gcn[None, :]
    nk1 = sw1_t.shape[0]
    nk2 = sw2_t.shape[0]
    ng1 = g1_t.shape[0]
    ng2 = g2_t.shape[0]

    out = pl.pallas_call(
        _mlp_body,
        grid=(n // blk,),
        in_specs=[
            pl.BlockSpec((_NC, blk, hid), lambda i: (0, i, 0)),
            pl.BlockSpec((blk, hid), lambda i: (i, 0)),
            pl.BlockSpec((blk, _NW), lambda i: (i, 0)),
            pl.BlockSpec((1, hid), lambda i: (0, 0)),
            pl.BlockSpec((ng1, hid), lambda i: (0, 0)),
            pl.BlockSpec((mid, hid), lambda i: (0, 0)),
            pl.BlockSpec((nk1, mid, hid), lambda i: (0, 0, 0)),
            pl.BlockSpec((mid, hid), lambda i: (0, 0)),
            pl.BlockSpec((ng2, mid), lambda i: (0, 0)),
            pl.BlockSpec((out_ch, mid), lambda i: (0, 0)),
            pl.BlockSpec((nk2, out_ch, mid), lambda i: (0, 0, 0)),
            pl.BlockSpec((out_ch, mid), lambda i: (0, 0)),
        ],
        out_specs=pl.BlockSpec((blk, out_ch), lambda i: (i, 0)),
        out_shape=jax.ShapeDtypeStruct((n, out_ch), jnp.float32),
    )(aggp, hs, degp, b2d, g1_t, base_w1, sw1_t, scaler1,
      g2_t, base_w2, sw2_t, scaler2)

    return out


# speedup vs baseline: 36.7672x; 1.0625x over previous
import functools

import jax
import jax.numpy as jnp
from jax import lax
from jax.experimental import pallas as pl
from jax.experimental.pallas import tpu as pltpu
from jax.experimental.pallas import tpu_sc as plsc

_NC = 2
_NS = 16
_NW = _NC * _NS
_L = 16

_SPLINE_ORDER = 3
_EDGE_CHUNK = 80


def _make_deg_kernel(n_nodes, n_edges):
    rows_pw = n_edges // _NW // _L
    nrow = n_nodes // _L
    mesh = plsc.VectorSubcoreMesh(core_axis_name="c", subcore_axis_name="s")

    @functools.partial(
        pl.kernel,
        mesh=mesh,
        out_type=jax.ShapeDtypeStruct((_NW, nrow, _L), jnp.float32),
        compiler_params=pltpu.CompilerParams(needs_layout_passes=False, use_tc_tiling_on_sc=False),
        scratch_types=[
            pltpu.VMEM((rows_pw, _L), jnp.int32),
            pltpu.VMEM((nrow, _L), jnp.float32),
        ],
    )
    def deg_kernel(dst_hbm, out_hbm, idx_v, hist_v):
        c = lax.axis_index("c")
        s = lax.axis_index("s")
        w = c * _NS + s

        zeros16 = jnp.zeros((_L,), jnp.float32)

        def zero_body(i, carry):
            hist_v[i] = zeros16
            return carry

        lax.fori_loop(0, nrow, zero_body, 0)

        pltpu.sync_copy(dst_hbm.at[w], idx_v)

        ones16 = jnp.ones((_L,), jnp.float32)

        def body(i, carry):
            idx = idx_v[i]
            plsc.addupdate_scatter(
                hist_v,
                [lax.shift_right_logical(idx, 4), lax.bitwise_and(idx, 15)],
                ones16)
            return carry

        lax.fori_loop(0, rows_pw, body, 0)

        pltpu.sync_copy(hist_v, out_hbm.at[w])

    return deg_kernel


_NBUF = 5


def _make_agg_kernel(n_nodes, n_edges, hid):
    ch = _EDGE_CHUNK
    nch = n_edges // _NW // ch
    assert nch % _NBUF == 0
    ngrp = nch // _NBUF
    nstage = 10
    rows_ps = n_nodes // nstage
    mesh = plsc.VectorSubcoreMesh(core_axis_name="c", subcore_axis_name="s")

    @functools.partial(
        pl.kernel,
        mesh=mesh,
        out_type=jax.ShapeDtypeStruct((_NC, n_nodes, hid), jnp.float32),
        compiler_params=pltpu.CompilerParams(needs_layout_passes=False, use_tc_tiling_on_sc=False),
        scratch_types=[
            pltpu.VMEM((nch, ch), jnp.int32),
            pltpu.VMEM((nch, ch), jnp.int32),
            pltpu.VMEM((_NBUF, ch, hid), jnp.float32),
            pltpu.VMEM_SHARED((n_nodes, hid), jnp.float32),
        ] + [pltpu.SemaphoreType.DMA] * (2 * _NBUF),
    )
    def agg_kernel(src_hbm, dst_hbm, hs_hbm, out_hbm, src_v, dst_v, rows_v,
                   acc_sh, *sems):
        gsem = sems[:_NBUF]
        ssem = sems[_NBUF:]
        c = lax.axis_index("c")
        s = lax.axis_index("s")
        w = c * _NS + s

        pltpu.sync_copy(src_hbm.at[w], src_v)
        pltpu.sync_copy(dst_hbm.at[w], dst_v)

        @pl.when(s < nstage)
        def _():
            pltpu.sync_copy(hs_hbm.at[pl.ds(s * rows_ps, rows_ps)],
                            acc_sh.at[pl.ds(s * rows_ps, rows_ps)])
        plsc.subcore_barrier()

        def wait_gather(b):
            pltpu.make_async_copy(hs_hbm.at[pl.ds(0, ch)], rows_v.at[b],
                                  gsem[b]).wait()

        def wait_scatter(b):
            pltpu.make_async_copy(rows_v.at[b], acc_sh.at[pl.ds(0, ch)],
                                  ssem[b]).wait()

        for b in range(_NBUF):
            pltpu.async_copy(hs_hbm.at[src_v.at[b]], rows_v.at[b], gsem[b])

        def body(t, carry):
            base = t * _NBUF
            for b in range(_NBUF):
                wait_gather(b)
                pltpu.async_copy(rows_v.at[b], acc_sh.at[dst_v.at[base + b]],
                                 ssem[b], add=True)
            for b in range(_NBUF):
                jn = base + _NBUF + b

                @pl.when(jn < nch)
                def _(b=b, jn=jn):
                    wait_scatter(b)
                    pltpu.async_copy(hs_hbm.at[src_v.at[jn]], rows_v.at[b],
                                     gsem[b])
            return carry

        lax.fori_loop(0, ngrp, body, 0)

        for b in range(_NBUF):
            wait_scatter(b)

        plsc.subcore_barrier()

        @pl.when(s < nstage)
        def _():
            pltpu.sync_copy(acc_sh.at[pl.ds(s * rows_ps, rows_ps)],
                            out_hbm.at[c, pl.ds(s * rows_ps, rows_ps)])

    return agg_kernel


def _hs_body(x_ref, w_ref, degp_ref, hs_ref):
    deg = jnp.sum(degp_ref[...], axis=1) + 1.0
    dinv = lax.rsqrt(deg)
    h = lax.dot_general(x_ref[...], w_ref[...], (((1,), (1,)), ((), ())),
                        preferred_element_type=jnp.float32)
    hs_ref[...] = h * dinv[:, None]


def _kan_apply(x, grid_t, bw, sw_t, sc):
    base = lax.dot_general(jax.nn.silu(x), bw, (((1,), (1,)), ((), ())),
                           preferred_element_type=jnp.float32)
    nb = grid_t.shape[0] - _SPLINE_ORDER - 1
    g0 = grid_t[0][None, :]
    h = grid_t[1][None, :] - g0
    t = (x - g0) / h
    m = jnp.floor(t)
    u = t - m
    c0 = u * u * u
    c1 = ((-3.0 * u + 3.0) * u + 3.0) * u + 1.0
    c2 = (3.0 * u - 6.0) * u * u + 4.0
    c3 = ((-u + 3.0) * u - 3.0) * u + 1.0
    scale = 1.0
    for k in range(1, _SPLINE_ORDER + 1):
        scale /= k
    out = base
    for j in range(nb):
        mj = m - float(j)
        val = jnp.where(
            mj == 0.0, c0,
            jnp.where(mj == 1.0, c1,
                      jnp.where(mj == 2.0, c2,
                                jnp.where(mj == 3.0, c3, 0.0))))
        sj = sw_t[j] * (sc * scale)
        out = out + lax.dot_general(val, sj, (((1,), (1,)), ((), ())),
                                    preferred_element_type=jnp.float32)
    return out


def _mlp_body(aggp_ref, hs_ref, degp_ref, b_ref, g1_ref, bw1_ref, sw1_ref,
              sc1_ref, g2_ref, bw2_ref, sw2_ref, sc2_ref, out_ref):
    deg = jnp.sum(degp_ref[...], axis=1) + 1.0
    dinv = lax.rsqrt(deg)
    agg = aggp_ref[0] + aggp_ref[1] - hs_ref[...]
    h1 = jnp.maximum(agg * dinv[:, None] + b_ref[...], 0.0)
    h2 = _kan_apply(h1, g1_ref[...], bw1_ref[...], sw1_ref[...], sc1_ref[...])
    h3 = _kan_apply(h2, g2_ref[...], bw2_ref[...], sw2_ref[...], sc2_ref[...])
    out_ref[...] = h3


def kernel(x, edge_index, W_gcn, b_gcn, grid1, base_w1, spline_w1, scaler1,
           grid2, base_w2, spline_w2, scaler2):
    n, in_ch = x.shape
    hid = W_gcn.shape[0]
    mid = base_w1.shape[0]
    out_ch = base_w2.shape[0]
    e = edge_index.shape[1]

    src = edge_index[0]
    dst = edge_index[1]

    degp = _make_deg_kernel(n, e)(
        dst.reshape(_NW, e // _NW // _L, _L)).reshape(_NW, n).T

    blk = 1000
    hs = pl.pallas_call(
        _hs_body,
        grid=(n // blk,),
        in_specs=[
            pl.BlockSpec((blk, in_ch), lambda i: (i, 0)),
            pl.BlockSpec((hid, in_ch), lambda i: (0, 0)),
            pl.BlockSpec((blk, _NW), lambda i: (i, 0)),
        ],
        out_specs=pl.BlockSpec((blk, hid), lambda i: (i, 0)),
        out_shape=jax.ShapeDtypeStruct((n, hid), jnp.float32),
    )(x, W_gcn, degp)

    ch = _EDGE_CHUNK
    nch = e // _NW // ch
    aggp = _make_agg_kernel(n, e, hid)(
        src.reshape(_NW, nch, ch), dst.reshape(_NW, nch, ch), hs)

    g1_t = grid1.T
    g2_t = grid2.T
    sw1_t = jnp.transpose(spline_w1, (2, 0, 1))
    sw2_t = jnp.transpose(spline_w2, (2, 0, 1))
    b2d = b_gcn[None, :]
    nk1 = sw1_t.shape[0]
    nk2 = sw2_t.shape[0]
    ng1 = g1_t.shape[0]
    ng2 = g2_t.shape[0]

    out = pl.pallas_call(
        _mlp_body,
        grid=(n // blk,),
        in_specs=[
            pl.BlockSpec((_NC, blk, hid), lambda i: (0, i, 0)),
            pl.BlockSpec((blk, hid), lambda i: (i, 0)),
            pl.BlockSpec((blk, _NW), lambda i: (i, 0)),
            pl.BlockSpec((1, hid), lambda i: (0, 0)),
            pl.BlockSpec((ng1, hid), lambda i: (0, 0)),
            pl.BlockSpec((mid, hid), lambda i: (0, 0)),
            pl.BlockSpec((nk1, mid, hid), lambda i: (0, 0, 0)),
            pl.BlockSpec((mid, hid), lambda i: (0, 0)),
            pl.BlockSpec((ng2, mid), lambda i: (0, 0)),
            pl.BlockSpec((out_ch, mid), lambda i: (0, 0)),
            pl.BlockSpec((nk2, out_ch, mid), lambda i: (0, 0, 0)),
            pl.BlockSpec((out_ch, mid), lambda i: (0, 0)),
        ],
        out_specs=pl.BlockSpec((blk, out_ch), lambda i: (i, 0)),
        out_shape=jax.ShapeDtypeStruct((n, out_ch), jnp.float32),
    )(aggp, hs, degp, b2d, g1_t, base_w1, sw1_t, scaler1,
      g2_t, base_w2, sw2_t, scaler2)

    return out


# speedup vs baseline: 43.6885x; 1.1882x over previous
import functools

import jax
import jax.numpy as jnp
from jax import lax
from jax.experimental import pallas as pl
from jax.experimental.pallas import tpu as pltpu
from jax.experimental.pallas import tpu_sc as plsc

_NC = 2
_NS = 16
_NW = _NC * _NS
_L = 16

_SPLINE_ORDER = 3
_EDGE_CHUNK = 80


def _make_deg_kernel(n_nodes, n_edges):
    epw = n_edges // _NW
    mesh = plsc.VectorSubcoreMesh(core_axis_name="c", subcore_axis_name="s")

    @functools.partial(
        pl.kernel,
        mesh=mesh,
        out_type=jax.ShapeDtypeStruct((_NW, n_nodes), jnp.float32),
        compiler_params=pltpu.CompilerParams(needs_layout_passes=False, use_tc_tiling_on_sc=False),
        scratch_types=[
            pltpu.VMEM((epw,), jnp.int32),
            pltpu.VMEM((n_nodes,), jnp.float32),
        ],
    )
    def deg_kernel(ei_hbm, out_hbm, idx_v, hist_v):
        c = lax.axis_index("c")
        s = lax.axis_index("s")
        w = c * _NS + s

        zeros16 = jnp.zeros((_L,), jnp.float32)

        def zero_body(i, carry):
            hist_v[pl.ds(i * _L, _L)] = zeros16
            return carry

        lax.fori_loop(0, n_nodes // _L, zero_body, 0)

        pltpu.sync_copy(ei_hbm.at[1, pl.ds(w * epw, epw)], idx_v)

        ones16 = jnp.ones((_L,), jnp.float32)

        def body(i, carry):
            idx = idx_v[pl.ds(i * _L, _L)]
            plsc.addupdate_scatter(hist_v, [idx], ones16)
            return carry

        lax.fori_loop(0, epw // _L, body, 0)

        pltpu.sync_copy(hist_v, out_hbm.at[w])

    return deg_kernel


_NBUF = 5


def _make_agg_kernel(n_nodes, n_edges, hid):
    ch = _EDGE_CHUNK
    nch = n_edges // _NW // ch
    assert nch % _NBUF == 0
    ngrp = nch // _NBUF
    nstage = 10
    rows_ps = n_nodes // nstage
    mesh = plsc.VectorSubcoreMesh(core_axis_name="c", subcore_axis_name="s")

    epw = n_edges // _NW

    @functools.partial(
        pl.kernel,
        mesh=mesh,
        out_type=jax.ShapeDtypeStruct((_NC, n_nodes, hid), jnp.float32),
        compiler_params=pltpu.CompilerParams(needs_layout_passes=False, use_tc_tiling_on_sc=False),
        scratch_types=[
            pltpu.VMEM((epw,), jnp.int32),
            pltpu.VMEM((epw,), jnp.int32),
            pltpu.VMEM((_NBUF, ch, hid), jnp.float32),
            pltpu.VMEM_SHARED((n_nodes, hid), jnp.float32),
        ] + [pltpu.SemaphoreType.DMA] * (2 * _NBUF),
    )
    def agg_kernel(ei_hbm, hs_hbm, out_hbm, src_v, dst_v, rows_v,
                   acc_sh, *sems):
        gsem = sems[:_NBUF]
        ssem = sems[_NBUF:]
        c = lax.axis_index("c")
        s = lax.axis_index("s")
        w = c * _NS + s

        pltpu.sync_copy(ei_hbm.at[0, pl.ds(w * epw, epw)], src_v)
        pltpu.sync_copy(ei_hbm.at[1, pl.ds(w * epw, epw)], dst_v)

        @pl.when(s < nstage)
        def _():
            pltpu.sync_copy(hs_hbm.at[pl.ds(s * rows_ps, rows_ps)],
                            acc_sh.at[pl.ds(s * rows_ps, rows_ps)])
        plsc.subcore_barrier()

        def wait_gather(b):
            pltpu.make_async_copy(hs_hbm.at[pl.ds(0, ch)], rows_v.at[b],
                                  gsem[b]).wait()

        def wait_scatter(b):
            pltpu.make_async_copy(rows_v.at[b], acc_sh.at[pl.ds(0, ch)],
                                  ssem[b]).wait()

        for b in range(_NBUF):
            pltpu.async_copy(hs_hbm.at[src_v.at[pl.ds(b * ch, ch)]],
                             rows_v.at[b], gsem[b])

        def body(t, carry):
            base = t * _NBUF
            for b in range(_NBUF):
                wait_gather(b)
                pltpu.async_copy(
                    rows_v.at[b],
                    acc_sh.at[dst_v.at[pl.ds((base + b) * ch, ch)]],
                    ssem[b], add=True)
            for b in range(_NBUF):
                jn = base + _NBUF + b

                @pl.when(jn < nch)
                def _(b=b, jn=jn):
                    wait_scatter(b)
                    pltpu.async_copy(hs_hbm.at[src_v.at[pl.ds(jn * ch, ch)]],
                                     rows_v.at[b], gsem[b])
            return carry

        lax.fori_loop(0, ngrp, body, 0)

        for b in range(_NBUF):
            wait_scatter(b)

        plsc.subcore_barrier()

        @pl.when(s < nstage)
        def _():
            pltpu.sync_copy(acc_sh.at[pl.ds(s * rows_ps, rows_ps)],
                            out_hbm.at[c, pl.ds(s * rows_ps, rows_ps)])

    return agg_kernel


def _hs_body(x_ref, w_ref, degp_ref, hs_ref, dinvb_ref):
    hid = hs_ref.shape[1]
    ones = jnp.ones((degp_ref.shape[0], hid), jnp.float32)
    degb = lax.dot_general(degp_ref[...], ones, (((0,), (0,)), ((), ())),
                           preferred_element_type=jnp.float32)
    dinvb = lax.rsqrt(degb + 1.0)
    h = lax.dot_general(x_ref[...], w_ref[...], (((1,), (1,)), ((), ())),
                        preferred_element_type=jnp.float32)
    hs_ref[...] = h * dinvb
    dinvb_ref[...] = dinvb


def _kan_apply(x, grid_t, bw, sw_t, sc):
    base = lax.dot_general(jax.nn.silu(x), bw, (((1,), (1,)), ((), ())),
                           preferred_element_type=jnp.float32)
    nb = grid_t.shape[0] - _SPLINE_ORDER - 1
    g0 = grid_t[0][None, :]
    h = grid_t[1][None, :] - g0
    t = (x - g0) / h
    m = jnp.floor(t)
    u = t - m
    c0 = u * u * u
    c1 = ((-3.0 * u + 3.0) * u + 3.0) * u + 1.0
    c2 = (3.0 * u - 6.0) * u * u + 4.0
    c3 = ((-u + 3.0) * u - 3.0) * u + 1.0
    scale = 1.0
    for k in range(1, _SPLINE_ORDER + 1):
        scale /= k
    out = base
    for j in range(nb):
        mj = m - float(j)
        val = jnp.where(
            mj == 0.0, c0,
            jnp.where(mj == 1.0, c1,
                      jnp.where(mj == 2.0, c2,
                                jnp.where(mj == 3.0, c3, 0.0))))
        sj = sw_t[j] * (sc * scale)
        out = out + lax.dot_general(val, sj, (((1,), (1,)), ((), ())),
                                    preferred_element_type=jnp.float32)
    return out


def _mlp_body(aggp_ref, hs_ref, dinvb_ref, b_ref, g1_ref, bw1_ref, sw1_ref,
              sc1_ref, g2_ref, bw2_ref, sw2_ref, sc2_ref, out_ref):
    agg = aggp_ref[0] + aggp_ref[1] - hs_ref[...]
    h1 = jnp.maximum(agg * dinvb_ref[...] + b_ref[...], 0.0)
    h2 = _kan_apply(h1, g1_ref[...], bw1_ref[...], sw1_ref[...], sc1_ref[...])
    h3 = _kan_apply(h2, g2_ref[...], bw2_ref[...], sw2_ref[...], sc2_ref[...])
    out_ref[...] = h3


def kernel(x, edge_index, W_gcn, b_gcn, grid1, base_w1, spline_w1, scaler1,
           grid2, base_w2, spline_w2, scaler2):
    n, in_ch = x.shape
    hid = W_gcn.shape[0]
    mid = base_w1.shape[0]
    out_ch = base_w2.shape[0]
    e = edge_index.shape[1]

    degp = _make_deg_kernel(n, e)(edge_index)

    hs, dinvb = pl.pallas_call(
        _hs_body,
        out_shape=[jax.ShapeDtypeStruct((n, hid), jnp.float32),
                   jax.ShapeDtypeStruct((n, hid), jnp.float32)],
    )(x, W_gcn, degp)

    aggp = _make_agg_kernel(n, e, hid)(edge_index, hs)

    g1_t = grid1.T
    g2_t = grid2.T
    sw1_t = jnp.transpose(spline_w1, (2, 0, 1))
    sw2_t = jnp.transpose(spline_w2, (2, 0, 1))
    b2d = b_gcn[None, :]
    nk1 = sw1_t.shape[0]
    nk2 = sw2_t.shape[0]
    ng1 = g1_t.shape[0]
    ng2 = g2_t.shape[0]

    blk = 1000
    out = pl.pallas_call(
        _mlp_body,
        grid=(n // blk,),
        in_specs=[
            pl.BlockSpec((_NC, blk, hid), lambda i: (0, i, 0)),
            pl.BlockSpec((blk, hid), lambda i: (i, 0)),
            pl.BlockSpec((blk, hid), lambda i: (i, 0)),
            pl.BlockSpec((1, hid), lambda i: (0, 0)),
            pl.BlockSpec((ng1, hid), lambda i: (0, 0)),
            pl.BlockSpec((mid, hid), lambda i: (0, 0)),
            pl.BlockSpec((nk1, mid, hid), lambda i: (0, 0, 0)),
            pl.BlockSpec((mid, hid), lambda i: (0, 0)),
            pl.BlockSpec((ng2, mid), lambda i: (0, 0)),
            pl.BlockSpec((out_ch, mid), lambda i: (0, 0)),
            pl.BlockSpec((nk2, out_ch, mid), lambda i: (0, 0, 0)),
            pl.BlockSpec((out_ch, mid), lambda i: (0, 0)),
        ],
        out_specs=pl.BlockSpec((blk, out_ch), lambda i: (i, 0)),
        out_shape=jax.ShapeDtypeStruct((n, out_ch), jnp.float32),
    )(aggp, hs, dinvb, b2d, g1_t, base_w1, sw1_t, scaler1,
      g2_t, base_w2, sw2_t, scaler2)

    return out


# speedup vs baseline: 51.9420x; 1.1889x over previous
import functools

import jax
import jax.numpy as jnp
from jax import lax
from jax.experimental import pallas as pl
from jax.experimental.pallas import tpu as pltpu
from jax.experimental.pallas import tpu_sc as plsc

_NC = 2
_NS = 16
_NW = _NC * _NS
_L = 16

_SPLINE_ORDER = 3
_EDGE_CHUNK = 80


def _make_deg_kernel(n_nodes, n_edges):
    epw = n_edges // _NW
    mesh = plsc.VectorSubcoreMesh(core_axis_name="c", subcore_axis_name="s")

    @functools.partial(
        pl.kernel,
        mesh=mesh,
        out_type=jax.ShapeDtypeStruct((_NW, n_nodes), jnp.float32),
        compiler_params=pltpu.CompilerParams(needs_layout_passes=False, use_tc_tiling_on_sc=False),
        scratch_types=[
            pltpu.VMEM((epw,), jnp.int32),
            pltpu.VMEM((n_nodes,), jnp.float32),
        ],
    )
    def deg_kernel(ei_hbm, out_hbm, idx_v, hist_v):
        c = lax.axis_index("c")
        s = lax.axis_index("s")
        w = c * _NS + s

        zeros16 = jnp.zeros((_L,), jnp.float32)

        def zero_body(i, carry):
            hist_v[pl.ds(i * _L, _L)] = zeros16
            return carry

        lax.fori_loop(0, n_nodes // _L, zero_body, 0)

        pltpu.sync_copy(ei_hbm.at[1, pl.ds(w * epw, epw)], idx_v)

        ones16 = jnp.ones((_L,), jnp.float32)

        def body(i, carry):
            idx = idx_v[pl.ds(i * _L, _L)]
            plsc.addupdate_scatter(hist_v, [idx], ones16)
            return carry

        lax.fori_loop(0, epw // _L, body, 0)

        pltpu.sync_copy(hist_v, out_hbm.at[w])

    return deg_kernel


_NBUF = 5


def _make_agg_kernel(n_nodes, n_edges, hid):
    ch = _EDGE_CHUNK
    nch = n_edges // _NW // ch
    assert nch % _NBUF == 0
    ngrp = nch // _NBUF
    nstage = 10
    rows_ps = n_nodes // nstage
    mesh = plsc.VectorSubcoreMesh(core_axis_name="c", subcore_axis_name="s")

    epw = n_edges // _NW

    @functools.partial(
        pl.kernel,
        mesh=mesh,
        out_type=jax.ShapeDtypeStruct((_NC, n_nodes, hid), jnp.float32),
        compiler_params=pltpu.CompilerParams(needs_layout_passes=False, use_tc_tiling_on_sc=False),
        scratch_types=[
            pltpu.VMEM((epw,), jnp.int32),
            pltpu.VMEM((epw,), jnp.int32),
            pltpu.VMEM((_NBUF, ch, hid), jnp.float32),
            pltpu.VMEM_SHARED((n_nodes, hid), jnp.float32),
        ] + [pltpu.SemaphoreType.DMA] * (2 * _NBUF),
    )
    def agg_kernel(ei_hbm, hs_hbm, out_hbm, src_v, dst_v, rows_v,
                   acc_sh, *sems):
        gsem = sems[:_NBUF]
        ssem = sems[_NBUF:]
        c = lax.axis_index("c")
        s = lax.axis_index("s")
        w = c * _NS + s

        pltpu.sync_copy(ei_hbm.at[0, pl.ds(w * epw, epw)], src_v)
        pltpu.sync_copy(ei_hbm.at[1, pl.ds(w * epw, epw)], dst_v)

        @pl.when(s < nstage)
        def _():
            pltpu.sync_copy(hs_hbm.at[pl.ds(s * rows_ps, rows_ps)],
                            acc_sh.at[pl.ds(s * rows_ps, rows_ps)])
        plsc.subcore_barrier()

        def wait_gather(b):
            pltpu.make_async_copy(hs_hbm.at[pl.ds(0, ch)], rows_v.at[b],
                                  gsem[b]).wait()

        def wait_scatter(b):
            pltpu.make_async_copy(rows_v.at[b], acc_sh.at[pl.ds(0, ch)],
                                  ssem[b]).wait()

        for b in range(_NBUF):
            pltpu.async_copy(hs_hbm.at[src_v.at[pl.ds(b * ch, ch)]],
                             rows_v.at[b], gsem[b])

        def body(t, carry):
            base = t * _NBUF
            for b in range(_NBUF):
                wait_gather(b)
                pltpu.async_copy(
                    rows_v.at[b],
                    acc_sh.at[dst_v.at[pl.ds((base + b) * ch, ch)]],
                    ssem[b], add=True)
            for b in range(_NBUF):
                jn = base + _NBUF + b

                @pl.when(jn < nch)
                def _(b=b, jn=jn):
                    wait_scatter(b)
                    pltpu.async_copy(hs_hbm.at[src_v.at[pl.ds(jn * ch, ch)]],
                                     rows_v.at[b], gsem[b])
            return carry

        lax.fori_loop(0, ngrp, body, 0)

        for b in range(_NBUF):
            wait_scatter(b)

        plsc.subcore_barrier()

        @pl.when(s < nstage)
        def _():
            pltpu.sync_copy(acc_sh.at[pl.ds(s * rows_ps, rows_ps)],
                            out_hbm.at[c, pl.ds(s * rows_ps, rows_ps)])

    return agg_kernel


def _hs_body(x_ref, w_ref, degp_ref, hs_ref, dinvb_ref):
    hid = hs_ref.shape[1]
    ones = jnp.ones((degp_ref.shape[0], hid), jnp.float32)
    degb = lax.dot_general(degp_ref[...], ones, (((0,), (0,)), ((), ())),
                           preferred_element_type=jnp.float32)
    dinvb = lax.rsqrt(degb + 1.0)
    h = lax.dot_general(x_ref[...], w_ref[...], (((1,), (1,)), ((), ())),
                        preferred_element_type=jnp.float32)
    hs_ref[...] = h * dinvb
    dinvb_ref[...] = dinvb


def _kan_apply(xT, grid_t, bw, sw_t, sc):
    g0 = grid_t[0][:, None]
    gh = grid_t[1][:, None] - g0
    t = (xT - g0) / gh
    m = jnp.floor(t)
    u = t - m
    c0 = u * u * u
    c1 = ((-3.0 * u + 3.0) * u + 3.0) * u + 1.0
    c2 = (3.0 * u - 6.0) * u * u + 4.0
    c3 = ((-u + 3.0) * u - 3.0) * u + 1.0
    scale = 1.0
    for k in range(1, _SPLINE_ORDER + 1):
        scale /= k
    nb = grid_t.shape[0] - _SPLINE_ORDER - 1
    outT = lax.dot_general(bw, jax.nn.silu(xT), (((1,), (0,)), ((), ())),
                           preferred_element_type=jnp.float32)
    for j in range(nb):
        mj = m - float(j)
        val = jnp.where(
            mj == 0.0, c0,
            jnp.where(mj == 1.0, c1,
                      jnp.where(mj == 2.0, c2,
                                jnp.where(mj == 3.0, c3, 0.0))))
        sj = sw_t[j] * (sc * scale)
        outT = outT + lax.dot_general(sj, val, (((1,), (0,)), ((), ())),
                                      preferred_element_type=jnp.float32)
    return outT


def _mlp_body(aggp_ref, hs_ref, dinvb_ref, b_ref, g1_ref, bw1_ref, sw1_ref,
              sc1_ref, g2_ref, bw2_ref, sw2_ref, sc2_ref, out_ref):
    agg = aggp_ref[0] + aggp_ref[1] - hs_ref[...]
    h1 = jnp.maximum(agg * dinvb_ref[...] + b_ref[...], 0.0)
    h1T = lax.transpose(h1, (1, 0))
    h2T = _kan_apply(h1T, g1_ref[...], bw1_ref[...], sw1_ref[...],
                     sc1_ref[...])
    h3T = _kan_apply(h2T, g2_ref[...], bw2_ref[...], sw2_ref[...],
                     sc2_ref[...])
    out_ref[...] = h3T[None]


def kernel(x, edge_index, W_gcn, b_gcn, grid1, base_w1, spline_w1, scaler1,
           grid2, base_w2, spline_w2, scaler2):
    n, in_ch = x.shape
    hid = W_gcn.shape[0]
    mid = base_w1.shape[0]
    out_ch = base_w2.shape[0]
    e = edge_index.shape[1]

    degp = _make_deg_kernel(n, e)(edge_index)

    hs, dinvb = pl.pallas_call(
        _hs_body,
        out_shape=[jax.ShapeDtypeStruct((n, hid), jnp.float32),
                   jax.ShapeDtypeStruct((n, hid), jnp.float32)],
    )(x, W_gcn, degp)

    aggp = _make_agg_kernel(n, e, hid)(edge_index, hs)

    g1_t = grid1.T
    g2_t = grid2.T
    sw1_t = jnp.transpose(spline_w1, (2, 0, 1))
    sw2_t = jnp.transpose(spline_w2, (2, 0, 1))
    b2d = b_gcn[None, :]
    nk1 = sw1_t.shape[0]
    nk2 = sw2_t.shape[0]
    ng1 = g1_t.shape[0]
    ng2 = g2_t.shape[0]

    blk = 2000
    out = pl.pallas_call(
        _mlp_body,
        grid=(n // blk,),
        in_specs=[
            pl.BlockSpec((_NC, blk, hid), lambda i: (0, i, 0)),
            pl.BlockSpec((blk, hid), lambda i: (i, 0)),
            pl.BlockSpec((blk, hid), lambda i: (i, 0)),
            pl.BlockSpec((1, hid), lambda i: (0, 0)),
            pl.BlockSpec((ng1, hid), lambda i: (0, 0)),
            pl.BlockSpec((mid, hid), lambda i: (0, 0)),
            pl.BlockSpec((nk1, mid, hid), lambda i: (0, 0, 0)),
            pl.BlockSpec((mid, hid), lambda i: (0, 0)),
            pl.BlockSpec((ng2, mid), lambda i: (0, 0)),
            pl.BlockSpec((out_ch, mid), lambda i: (0, 0)),
            pl.BlockSpec((nk2, out_ch, mid), lambda i: (0, 0, 0)),
            pl.BlockSpec((out_ch, mid), lambda i: (0, 0)),
        ],
        out_specs=pl.BlockSpec((1, out_ch, blk), lambda i: (i, 0, 0)),
        out_shape=jax.ShapeDtypeStruct((n // blk, out_ch, blk), jnp.float32),
    )(aggp, hs, dinvb, b2d, g1_t, base_w1, sw1_t, scaler1,
      g2_t, base_w2, sw2_t, scaler2)

    return out.reshape(n, out_ch)
